# baseline (speedup 1.0000x reference)
import jax
import jax.numpy as jnp
from jax import lax
from jax.experimental import pallas as pl
from jax.experimental.pallas import tpu as pltpu, tpu_sc as plsc

_RANK = 64
_BATCH = 16384
_NW = 32
_BPW = _BATCH // _NW
_CH = 32
_NCH = _BPW // _CH
_L = 16


def _dpr_body(users_hbm, items_hbm, utab_hbm, itab_hbm, w_hbm, b_hbm,
              mean_hbm, std_hbm, logvar_hbm,
              uidx, iidx, ubuf, ibuf,
              mean_v, std_v, logvar_v, w_v, b_v, sem):
    wid = lax.axis_index("s") * 2 + lax.axis_index("c")
    base = wid * _BPW

    pltpu.sync_copy(users_hbm.at[pl.ds(base, _BPW)], uidx)
    pltpu.sync_copy(items_hbm.at[pl.ds(base, _BPW)], iidx)
    pltpu.sync_copy(w_hbm, w_v)
    pltpu.sync_copy(b_hbm, b_v)

    wm = [w_v[0, pl.ds(k * _L, _L)] for k in range(_RANK // _L)]
    wlv = [w_v[1, pl.ds(k * _L, _L)] for k in range(_RANK // _L)]
    bm = b_v[0, pl.ds(0, _L)]
    blv = b_v[1, pl.ds(0, _L)]
    lane = lax.iota(jnp.int32, _L)
    seven = jnp.full((_L,), 7, jnp.int32)
    zero = jnp.zeros((_L,), jnp.float32)

    def chunk_step(ci, _):
        c0 = ci * _CH
        minus8 = jnp.full((_L,), -8, jnp.int32)
        copies = []
        for g in range(_CH // _L):
            uv = uidx[pl.ds(c0 + g * _L, _L)] & minus8
            iv = iidx[pl.ds(c0 + g * _L, _L)] & minus8
            for l in range(_L):
                slot = g * _L + l
                copies.append(pltpu.async_copy(
                    utab_hbm.at[pl.ds(pl.multiple_of(uv[l], 8), 8)],
                    ubuf.at[slot], sem))
                copies.append(pltpu.async_copy(
                    itab_hbm.at[pl.ds(pl.multiple_of(iv[l], 8), 8)],
                    ibuf.at[slot], sem))
        for c in copies:
            c.wait()

        for g in range(_CH // _L):
            b0 = c0 + g * _L
            su = uidx[pl.ds(b0, _L)] & seven
            si = iidx[pl.ds(b0, _L)] & seven
            accm = zero
            acclv = zero
            for r in range(_L):
                slot = g * _L + r
                am = None
                alv = None
                for k in range(_RANK // _L):
                    u = ubuf[slot, su[r], pl.ds(k * _L, _L)]
                    it = ibuf[slot, si[r], pl.ds(k * _L, _L)]
                    inter = u * it
                    tm = inter * wm[k]
                    tlv = inter * wlv[k]
                    am = tm if am is None else am + tm
                    alv = tlv if alv is None else alv + tlv
                sel = lane == r
                accm = jnp.where(sel, jnp.sum(am), accm)
                acclv = jnp.where(sel, jnp.sum(alv), acclv)
            lv = acclv + blv
            mean_v[pl.ds(b0, _L)] = accm + bm
            logvar_v[pl.ds(b0, _L)] = lv
            std_v[pl.ds(b0, _L)] = jnp.exp(0.5 * lv)
        return _

    lax.fori_loop(0, _NCH, chunk_step, 0)

    pltpu.sync_copy(mean_v, mean_hbm.at[pl.ds(base, _BPW)])
    pltpu.sync_copy(std_v, std_hbm.at[pl.ds(base, _BPW)])
    pltpu.sync_copy(logvar_v, logvar_hbm.at[pl.ds(base, _BPW)])


@jax.jit
def _dpr(users, items, utab, itab, w_cat, bv):
    mesh = plsc.VectorSubcoreMesh(core_axis_name="c", subcore_axis_name="s")
    out = jax.ShapeDtypeStruct((_BATCH,), jnp.float32)
    f = pl.kernel(
        _dpr_body,
        out_type=(out, out, out),
        mesh=mesh,
        scratch_types=[
            pltpu.VMEM((_BPW,), jnp.int32),
            pltpu.VMEM((_BPW,), jnp.int32),
            pltpu.VMEM((_CH, 8, _RANK), jnp.float32),
            pltpu.VMEM((_CH, 8, _RANK), jnp.float32),
            pltpu.VMEM((_BPW,), jnp.float32),
            pltpu.VMEM((_BPW,), jnp.float32),
            pltpu.VMEM((_BPW,), jnp.float32),
            pltpu.VMEM((2, _RANK), jnp.float32),
            pltpu.VMEM((2, _L), jnp.float32),
            pltpu.SemaphoreType.DMA,
        ],
        compiler_params=pltpu.CompilerParams(needs_layout_passes=False),
    )
    return f(users, items, utab, itab, w_cat, bv)


def kernel(users, items, user_table, item_table, W_mean, b_mean, W_logvar,
           b_logvar):
    w_cat = jnp.stack([W_mean.reshape(_RANK), W_logvar.reshape(_RANK)])
    bv = jnp.stack([jnp.full((_L,), b_mean[0], jnp.float32),
                    jnp.full((_L,), b_logvar[0], jnp.float32)])
    mean, std, logvar = _dpr(users, items, user_table, item_table, w_cat, bv)
    return (mean, std, logvar)
